# Initial kernel scaffold; baseline (speedup 1.0000x reference)
import jax
import jax.numpy as jnp
from jax.experimental import pallas as pl


def kernel(x, edge_index, W1, b1, W2, b2, W3, b3):
    raise NotImplementedError("write your pallas kernel here")


# speedup vs baseline: 31.5798x; 31.5798x over previous
import functools

import jax
import jax.numpy as jnp
from jax import lax
from jax.experimental import pallas as pl
from jax.experimental.pallas import tpu as pltpu
from jax.experimental.pallas import tpu_sc as plsc

N = 10000
E = 320000
F = 16
G = 125
NC = 2
NS = 16
NW = NC * NS
GPT = E // (NW * G)
NP = 10240
RPT = NP // NS

_MESH = plsc.VectorSubcoreMesh(core_axis_name="c", subcore_axis_name="s")


def _zero_acc_slice(zbuf, acc, s):
    def zb(i, carry):
        zbuf[i, :] = jnp.zeros((F,), jnp.float32)
        return carry
    lax.fori_loop(0, RPT, zb, 0)
    pltpu.sync_copy(zbuf, acc.at[pl.ds(s * RPT, RPT)])


def _copy_out(acc, out_hbm, c, s):
    pltpu.sync_copy(acc.at[pl.ds(s * RPT, RPT)],
                    out_hbm.at[c, pl.ds(s * RPT, RPT)])


@functools.partial(
    pl.kernel,
    out_type=jax.ShapeDtypeStruct((NC, NP, F), jnp.float32),
    mesh=_MESH,
    compiler_params=pltpu.CompilerParams(use_tc_tiling_on_sc=False),
    scratch_types=[
        pltpu.VMEM((GPT, G), jnp.int32),
        pltpu.VMEM((G, F), jnp.float32),
        pltpu.VMEM((RPT, F), jnp.float32),
        pltpu.VMEM_SHARED((NP, F), jnp.float32),
    ],
)
def _deg_pass(dst_hbm, out_hbm, dstv, ones_v, zbuf, acc):
    c = lax.axis_index("c")
    s = lax.axis_index("s")
    wid = c * NS + s
    _zero_acc_slice(zbuf, acc, s)

    def ob(i, carry):
        ones_v[i, :] = jnp.ones((F,), jnp.float32)
        return carry
    lax.fori_loop(0, G, ob, 0)
    pltpu.sync_copy(dst_hbm.at[pl.ds(wid * GPT, GPT)], dstv)
    plsc.subcore_barrier()

    def grp(j, carry):
        pltpu.sync_copy(ones_v, acc.at[dstv.at[j]], add=True)
        return carry
    lax.fori_loop(0, GPT, grp, 0)
    plsc.subcore_barrier()
    _copy_out(acc, out_hbm, c, s)


@functools.partial(
    pl.kernel,
    out_type=jax.ShapeDtypeStruct((NC, NP, F), jnp.float32),
    mesh=_MESH,
    compiler_params=pltpu.CompilerParams(use_tc_tiling_on_sc=False),
    scratch_types=[
        pltpu.VMEM((GPT, G), jnp.int32),
        pltpu.VMEM((GPT, G), jnp.int32),
        pltpu.VMEM((G, F), jnp.float32),
        pltpu.VMEM((RPT, F), jnp.float32),
        pltpu.VMEM_SHARED((NP, F), jnp.float32),
        pltpu.SemaphoreType.DMA,
    ],
)
def _prop_pass(z_hbm, src_hbm, dst_hbm, out_hbm, srcv, dstv, rows, zbuf, acc,
               sem):
    c = lax.axis_index("c")
    s = lax.axis_index("s")
    wid = c * NS + s
    _zero_acc_slice(zbuf, acc, s)
    pltpu.sync_copy(src_hbm.at[pl.ds(wid * GPT, GPT)], srcv)
    pltpu.sync_copy(dst_hbm.at[pl.ds(wid * GPT, GPT)], dstv)
    plsc.subcore_barrier()

    def grp(j, carry):
        pltpu.async_copy(z_hbm.at[srcv.at[j]], rows, sem).wait()
        pltpu.sync_copy(rows, acc.at[dstv.at[j]], add=True)
        return carry
    lax.fori_loop(0, GPT, grp, 0)
    plsc.subcore_barrier()
    _copy_out(acc, out_hbm, c, s)


def _d0_body(degp_ref, x_ref, w1_ref, z1_ref, dinv_ref):
    deg = degp_ref[0, :N] + degp_ref[1, :N]
    dinv = lax.rsqrt(deg[:, 0:1] + 1.0)
    dinv_ref[...] = dinv
    h = jnp.dot(x_ref[...], w1_ref[...], preferred_element_type=jnp.float32)
    z1_ref[...] = h * dinv


def _d1_body(p_ref, z_ref, dinv_ref, w_ref, b_ref, z2_ref):
    dinv = dinv_ref[...]
    h = jnp.maximum(dinv * (p_ref[0, :N] + p_ref[1, :N] + z_ref[...]) + b_ref[...],
                    0.0)
    z2_ref[...] = dinv * jnp.dot(h, w_ref[...],
                                 preferred_element_type=jnp.float32)


def _d2_body(p_ref, z_ref, dinv_ref, b_ref, z3_ref):
    dinv = dinv_ref[...]
    h = jnp.maximum(dinv * (p_ref[0, :N] + p_ref[1, :N] + z_ref[...]) + b_ref[...],
                    0.0)
    z3_ref[...] = dinv * h


def _d3_body(p_ref, z_ref, dinv_ref, w_ref, b_ref, out_ref):
    t = dinv_ref[...] * (p_ref[0, :N] + p_ref[1, :N] + z_ref[...])
    out_ref[...] = jnp.dot(t, w_ref[...],
                           preferred_element_type=jnp.float32) + b_ref[...]


def _f32(shape):
    return jax.ShapeDtypeStruct(shape, jnp.float32)


def kernel(x, edge_index, W1, b1, W2, b2, W3, b3):
    src = edge_index[0].astype(jnp.int32).reshape(E // G, G)
    dst = edge_index[1].astype(jnp.int32).reshape(E // G, G)

    degp = _deg_pass(dst)
    z1, dinv = pl.pallas_call(
        _d0_body, out_shape=[_f32((N, F)), _f32((N, 1))],
    )(degp, x, W1)

    p1 = _prop_pass(z1, src, dst)
    z2 = pl.pallas_call(_d1_body, out_shape=_f32((N, F)))(
        p1, z1, dinv, W2, b1.reshape(1, F))

    p2 = _prop_pass(z2, src, dst)
    z3 = pl.pallas_call(_d2_body, out_shape=_f32((N, F)))(
        p2, z2, dinv, b2.reshape(1, F))

    p3 = _prop_pass(z3, src, dst)
    out = pl.pallas_call(_d3_body, out_shape=_f32((N, W3.shape[1])))(
        p3, z3, dinv, W3, b3.reshape(1, W3.shape[1]))
    return out


# speedup vs baseline: 56.9288x; 1.8027x over previous
import functools

import jax
import jax.numpy as jnp
from jax import lax
from jax.experimental import pallas as pl
from jax.experimental.pallas import tpu as pltpu
from jax.experimental.pallas import tpu_sc as plsc

N = 10000
E = 320000
F = 16
G = 125
NC = 2
NS = 16
NW = NC * NS
GPT = E // (NW * G)
NP = 10240
RPT = NP // NS
D = 8
NBUF = 2 * D

_MESH = plsc.VectorSubcoreMesh(core_axis_name="c", subcore_axis_name="s")


def _zero_acc_slice(zbuf, acc, s):
    def zb(i, carry):
        zbuf[i, :] = jnp.zeros((F,), jnp.float32)
        return carry
    lax.fori_loop(0, RPT, zb, 0)
    pltpu.sync_copy(zbuf, acc.at[pl.ds(s * RPT, RPT)])


def _copy_out(acc, out_hbm, c, s):
    pltpu.sync_copy(acc.at[pl.ds(s * RPT, RPT)],
                    out_hbm.at[c, pl.ds(s * RPT, RPT)])


@functools.partial(
    pl.kernel,
    out_type=jax.ShapeDtypeStruct((NC, NP, F), jnp.float32),
    mesh=_MESH,
    compiler_params=pltpu.CompilerParams(use_tc_tiling_on_sc=False),
    scratch_types=[
        pltpu.VMEM((GPT, G), jnp.int32),
        pltpu.VMEM((G, F), jnp.float32),
        pltpu.VMEM((RPT, F), jnp.float32),
        pltpu.VMEM_SHARED((NP, F), jnp.float32),
        pltpu.SemaphoreType.DMA,
    ],
)
def _deg_pass(dst_hbm, out_hbm, dstv, ones_v, zbuf, acc, ssem):
    c = lax.axis_index("c")
    s = lax.axis_index("s")
    wid = c * NS + s
    _zero_acc_slice(zbuf, acc, s)

    def ob(i, carry):
        ones_v[i, :] = jnp.ones((F,), jnp.float32)
        return carry
    lax.fori_loop(0, G, ob, 0)
    pltpu.sync_copy(dst_hbm.at[pl.ds(wid * GPT, GPT)], dstv)
    plsc.subcore_barrier()

    def grp(j, carry):
        @pl.when(j >= D)
        def _():
            pltpu.make_async_copy(ones_v, acc.at[dstv.at[j - D]], ssem).wait()
        pltpu.async_copy(ones_v, acc.at[dstv.at[j]], ssem, add=True)
        return carry
    lax.fori_loop(0, GPT, grp, 0)
    for i in range(D):
        pltpu.make_async_copy(ones_v, acc.at[dstv.at[GPT - D + i]],
                              ssem).wait()
    plsc.subcore_barrier()
    _copy_out(acc, out_hbm, c, s)


@functools.partial(
    pl.kernel,
    out_type=jax.ShapeDtypeStruct((NC, NP, F), jnp.float32),
    mesh=_MESH,
    compiler_params=pltpu.CompilerParams(use_tc_tiling_on_sc=False),
    scratch_types=[
        pltpu.VMEM((GPT, G), jnp.int32),
        pltpu.VMEM((GPT, G), jnp.int32),
        pltpu.VMEM((NBUF, G, F), jnp.float32),
        pltpu.VMEM((RPT, F), jnp.float32),
        pltpu.VMEM_SHARED((NP, F), jnp.float32),
        pltpu.SemaphoreType.DMA,
        pltpu.SemaphoreType.DMA,
    ],
)
def _prop_pass(z_hbm, src_hbm, dst_hbm, out_hbm, srcv, dstv, rows, zbuf, acc,
               gsem, ssem):
    c = lax.axis_index("c")
    s = lax.axis_index("s")
    wid = c * NS + s
    _zero_acc_slice(zbuf, acc, s)
    pltpu.sync_copy(src_hbm.at[pl.ds(wid * GPT, GPT)], srcv)
    pltpu.sync_copy(dst_hbm.at[pl.ds(wid * GPT, GPT)], dstv)
    plsc.subcore_barrier()

    for b in range(D):
        pltpu.async_copy(z_hbm.at[srcv.at[b]], rows.at[b], gsem)

    def grp(j, carry):
        jb = lax.rem(j, NBUF)

        @pl.when(j >= D)
        def _():
            jd = j - D
            pltpu.make_async_copy(rows.at[lax.rem(jd, NBUF)],
                                  acc.at[dstv.at[jd]], ssem).wait()
        pltpu.make_async_copy(z_hbm.at[srcv.at[j]], rows.at[jb], gsem).wait()
        pltpu.async_copy(rows.at[jb], acc.at[dstv.at[j]], ssem, add=True)

        @pl.when(j + D < GPT)
        def _():
            jn = j + D
            pltpu.async_copy(z_hbm.at[srcv.at[jn]], rows.at[lax.rem(jn, NBUF)],
                             gsem)
        return carry
    lax.fori_loop(0, GPT, grp, 0)
    for i in range(D):
        jd = GPT - D + i
        pltpu.make_async_copy(rows.at[jd % NBUF], acc.at[dstv.at[jd]],
                              ssem).wait()
    plsc.subcore_barrier()
    _copy_out(acc, out_hbm, c, s)


def _d0_body(degp_ref, x_ref, w1_ref, z1_ref, dinv_ref):
    deg = degp_ref[0, :N] + degp_ref[1, :N]
    dinv = lax.rsqrt(deg[:, 0:1] + 1.0)
    dinv_ref[...] = dinv
    h = jnp.dot(x_ref[...], w1_ref[...], preferred_element_type=jnp.float32)
    z1_ref[...] = h * dinv


def _d1_body(p_ref, z_ref, dinv_ref, w_ref, b_ref, z2_ref):
    dinv = dinv_ref[...]
    h = jnp.maximum(dinv * (p_ref[0, :N] + p_ref[1, :N] + z_ref[...]) + b_ref[...],
                    0.0)
    z2_ref[...] = dinv * jnp.dot(h, w_ref[...],
                                 preferred_element_type=jnp.float32)


def _d2_body(p_ref, z_ref, dinv_ref, b_ref, z3_ref):
    dinv = dinv_ref[...]
    h = jnp.maximum(dinv * (p_ref[0, :N] + p_ref[1, :N] + z_ref[...]) + b_ref[...],
                    0.0)
    z3_ref[...] = dinv * h


def _d3_body(p_ref, z_ref, dinv_ref, w_ref, b_ref, out_ref):
    t = dinv_ref[...] * (p_ref[0, :N] + p_ref[1, :N] + z_ref[...])
    out_ref[...] = jnp.dot(t, w_ref[...],
                           preferred_element_type=jnp.float32) + b_ref[...]


def _f32(shape):
    return jax.ShapeDtypeStruct(shape, jnp.float32)


def kernel(x, edge_index, W1, b1, W2, b2, W3, b3):
    src = edge_index[0].astype(jnp.int32).reshape(E // G, G)
    dst = edge_index[1].astype(jnp.int32).reshape(E // G, G)

    degp = _deg_pass(dst)
    z1, dinv = pl.pallas_call(
        _d0_body, out_shape=[_f32((N, F)), _f32((N, 1))],
    )(degp, x, W1)

    p1 = _prop_pass(z1, src, dst)
    z2 = pl.pallas_call(_d1_body, out_shape=_f32((N, F)))(
        p1, z1, dinv, W2, b1.reshape(1, F))

    p2 = _prop_pass(z2, src, dst)
    z3 = pl.pallas_call(_d2_body, out_shape=_f32((N, F)))(
        p2, z2, dinv, b2.reshape(1, F))

    p3 = _prop_pass(z3, src, dst)
    out = pl.pallas_call(_d3_body, out_shape=_f32((N, W3.shape[1])))(
        p3, z3, dinv, W3, b3.reshape(1, W3.shape[1]))
    return out
